# baseline (speedup 1.0000x reference)
import functools

import jax
import jax.numpy as jnp
from jax import lax
from jax.experimental import pallas as pl
from jax.experimental.pallas import tpu as pltpu
from jax.experimental.pallas import tpu_sc as plsc

_B = 16384
_D = 128
_L = 16
_C = 64
_NC = 2
_NS = 16
_NW = _NC * _NS


def _vsqrt(x):
    xc = jnp.maximum(x, jnp.float32(1e-30))
    bits = plsc.bitcast(xc, jnp.int32)
    seed = plsc.bitcast((bits >> 1) + jnp.int32(0x1FBD1DF5), jnp.float32)
    y = seed
    for _ in range(3):
        y = jnp.float32(0.5) * (y + xc / y)
    return y


@functools.lru_cache(maxsize=1)
def _build():
    bpw = _B // _NW
    nch = bpw // _C
    nsl = _D // _L
    mesh = plsc.VectorSubcoreMesh(
        core_axis_name="c", subcore_axis_name="s",
        num_cores=_NC, num_subcores=_NS)

    def _chunk_scratch():
        return [
            pltpu.VMEM((2 * _C, _D), jnp.float32),
            pltpu.VMEM((2 * _C, _D), jnp.float32),
            pltpu.VMEM((_C, _D), jnp.float32),
            pltpu.VMEM((_C, _D), jnp.float32),
            pltpu.VMEM((_C,), jnp.float32),
            pltpu.SemaphoreType.DMA,
            pltpu.SemaphoreType.DMA,
        ]

    @functools.partial(
        pl.kernel,
        mesh=mesh,
        out_type=jax.ShapeDtypeStruct((_B,), jnp.float32),
        scratch_types=[
            _chunk_scratch(),
            _chunk_scratch(),
            pltpu.VMEM((2 * bpw,), jnp.int32),
            pltpu.VMEM((bpw,), jnp.int32),
            pltpu.SemaphoreType.DMA,
        ],
        compiler_params=pltpu.CompilerParams(needs_layout_passes=False),
    )
    def trans_d(idx_ht, rels, ee, ep, ret, rpt, out,
                set0, set1, idxc, idxr, isem):
        wid = lax.axis_index("s") * _NC + lax.axis_index("c")
        base = wid * bpw
        lane = lax.iota(jnp.int32, _L)
        sets = (set0, set1)

        pltpu.async_copy(idx_ht.at[pl.ds(2 * base, 2 * bpw)], idxc, isem)
        pltpu.async_copy(rels.at[pl.ds(base, bpw)], idxr, isem)
        pltpu.make_async_copy(idx_ht.at[pl.ds(0, 2 * bpw)], idxc, isem).wait()
        pltpu.make_async_copy(rels.at[pl.ds(0, bpw)], idxr, isem).wait()

        def issue(buf, loc):
            het, hpt, re, rp, scores, sem, osem = buf
            iht = idxc.at[pl.ds(2 * loc, 2 * _C)]
            ir = idxr.at[pl.ds(loc, _C)]
            pltpu.async_copy(ee.at[iht], het, sem)
            pltpu.async_copy(ep.at[iht], hpt, sem)
            pltpu.async_copy(ret.at[ir], re, sem)
            pltpu.async_copy(rpt.at[ir], rp, sem)

        def drain(buf):
            het, hpt, re, rp, scores, sem, osem = buf
            for dst in (het, hpt):
                pltpu.make_async_copy(ee.at[pl.ds(0, 2 * _C)], dst, sem).wait()
            for dst in (re, rp):
                pltpu.make_async_copy(ee.at[pl.ds(0, _C)], dst, sem).wait()

        def drain_out(buf):
            het, hpt, re, rp, scores, sem, osem = buf
            pltpu.make_async_copy(out.at[pl.ds(base, _C)], scores, osem).wait()

        def compute(buf, loc):
            het, hpt, re, rp, scores, sem, osem = buf

            def group_body(g, carry2):
                vec = jnp.zeros((_L,), jnp.float32)
                for k in range(_L):
                    i = g * _L + k
                    sl = pl.ds(0, _L)
                    acch = hpt[i, sl] * het[i, sl]
                    acct = hpt[_C + i, sl] * het[_C + i, sl]
                    for j in range(1, nsl):
                        sl = pl.ds(j * _L, _L)
                        acch = acch + hpt[i, sl] * het[i, sl]
                        acct = acct + hpt[_C + i, sl] * het[_C + i, sl]
                    s = jnp.sum(acch) - jnp.sum(acct)
                    sl = pl.ds(0, _L)
                    v = het[i, sl] - het[_C + i, sl] + re[i, sl] + s * rp[i, sl]
                    nsq = v * v
                    for j in range(1, nsl):
                        sl = pl.ds(j * _L, _L)
                        v = het[i, sl] - het[_C + i, sl] + re[i, sl] + s * rp[i, sl]
                        nsq = nsq + v * v
                    vec = jnp.where(lane == k, jnp.sum(nsq), vec)
                scores[pl.ds(g * _L, _L)] = _vsqrt(vec)
                return carry2

            lax.fori_loop(0, _C // _L, group_body, 0, unroll=2)
            pltpu.async_copy(scores, out.at[pl.ds(base + loc, _C)], osem)

        issue(sets[0], 0)

        def pair_body(m, carry):
            loc0 = (2 * m) * _C
            drain(sets[0])
            issue(sets[1], loc0 + _C)

            @pl.when(m > 0)
            def _():
                drain_out(sets[0])

            compute(sets[0], loc0)
            drain(sets[1])

            @pl.when(m < nch // 2 - 1)
            def _():
                issue(sets[0], loc0 + 2 * _C)

            @pl.when(m > 0)
            def _():
                drain_out(sets[1])

            compute(sets[1], loc0 + _C)
            return carry

        lax.fori_loop(0, nch // 2, pair_body, 0)
        drain_out(sets[0])
        drain_out(sets[1])

    return trans_d


def kernel(heads, relations, tails, entity_embeddings, entity_projections,
           relation_embeddings, relation_projections):
    h = heads.astype(jnp.int32).reshape(-1, _C)
    t = tails.astype(jnp.int32).reshape(-1, _C)
    idx_ht = jnp.concatenate([h, t], axis=1).reshape(-1)
    k = _build()
    return k(
        idx_ht,
        relations.astype(jnp.int32),
        entity_embeddings,
        entity_projections,
        relation_embeddings,
        relation_projections,
    )


# speedup vs baseline: 1.7124x; 1.7124x over previous
import functools

import jax
import jax.numpy as jnp
from jax import lax
from jax.experimental import pallas as pl
from jax.experimental.pallas import tpu as pltpu
from jax.experimental.pallas import tpu_sc as plsc

_B = 16384
_D = 128
_L = 16
_C = 64
_NC = 2
_NS = 16
_NW = _NC * _NS


def _vsqrt(x):
    xc = jnp.maximum(x, jnp.float32(1e-30))
    bits = plsc.bitcast(xc, jnp.int32)
    seed = plsc.bitcast((bits >> 1) + jnp.int32(0x1FBD1DF5), jnp.float32)
    y = seed
    for _ in range(3):
        y = jnp.float32(0.5) * (y + xc / y)
    return y


@functools.lru_cache(maxsize=1)
def _build():
    bpw = _B // _NW
    nch = bpw // _C
    nsl = _D // _L
    mesh = plsc.VectorSubcoreMesh(
        core_axis_name="c", subcore_axis_name="s",
        num_cores=_NC, num_subcores=_NS)

    def _chunk_scratch():
        return [
            pltpu.VMEM((2 * _C, _D), jnp.float32),
            pltpu.VMEM((2 * _C, _D), jnp.float32),
            pltpu.VMEM((_C, _D), jnp.float32),
            pltpu.VMEM((_C, _D), jnp.float32),
            pltpu.VMEM((_C,), jnp.float32),
            pltpu.SemaphoreType.DMA,
            pltpu.SemaphoreType.DMA,
        ]

    @functools.partial(
        pl.kernel,
        mesh=mesh,
        out_type=jax.ShapeDtypeStruct((_B,), jnp.float32),
        scratch_types=[
            _chunk_scratch(),
            _chunk_scratch(),
            pltpu.VMEM((2 * bpw,), jnp.int32),
            pltpu.VMEM((bpw,), jnp.int32),
            pltpu.SemaphoreType.DMA,
        ],
        compiler_params=pltpu.CompilerParams(needs_layout_passes=False),
    )
    def trans_d(idx_ht, rels, ee, ep, ret, rpt, out,
                set0, set1, idxc, idxr, isem):
        wid = lax.axis_index("s") * _NC + lax.axis_index("c")
        base = wid * bpw
        lane = lax.iota(jnp.int32, _L)
        sets = (set0, set1)

        pltpu.async_copy(idx_ht.at[pl.ds(2 * base, 2 * bpw)], idxc, isem)
        pltpu.async_copy(rels.at[pl.ds(base, bpw)], idxr, isem)
        pltpu.make_async_copy(idx_ht.at[pl.ds(0, 2 * bpw)], idxc, isem).wait()
        pltpu.make_async_copy(rels.at[pl.ds(0, bpw)], idxr, isem).wait()

        def issue(buf, loc):
            het, hpt, re, rp, scores, sem, osem = buf
            iht = idxc.at[pl.ds(2 * loc, 2 * _C)]
            ir = idxr.at[pl.ds(loc, _C)]
            pltpu.async_copy(ee.at[iht], het, sem)
            pltpu.async_copy(ep.at[iht], hpt, sem)
            pltpu.async_copy(ret.at[ir], re, sem)
            pltpu.async_copy(rpt.at[ir], rp, sem)

        def drain(buf):
            het, hpt, re, rp, scores, sem, osem = buf
            for dst in (het, hpt):
                pltpu.make_async_copy(ee.at[pl.ds(0, 2 * _C)], dst, sem).wait()
            for dst in (re, rp):
                pltpu.make_async_copy(ee.at[pl.ds(0, _C)], dst, sem).wait()

        def drain_out(buf):
            het, hpt, re, rp, scores, sem, osem = buf
            pltpu.make_async_copy(out.at[pl.ds(base, _C)], scores, osem).wait()

        def compute(buf, loc):
            het, hpt, re, rp, scores, sem, osem = buf

            def group_body(g, carry2):
                vec = jnp.zeros((_L,), jnp.float32)
                for k in range(_L):
                    i = g * _L + k
                    sl = pl.ds(0, _L)
                    acch = hpt[i, sl] * het[i, sl]
                    acct = hpt[_C + i, sl] * het[_C + i, sl]
                    for j in range(1, nsl):
                        sl = pl.ds(j * _L, _L)
                        acch = acch + hpt[i, sl] * het[i, sl]
                        acct = acct + hpt[_C + i, sl] * het[_C + i, sl]
                    s = jnp.sum(acch) - jnp.sum(acct)
                    sl = pl.ds(0, _L)
                    v = het[i, sl] - het[_C + i, sl] + re[i, sl] + s * rp[i, sl]
                    nsq = v * v
                    for j in range(1, nsl):
                        sl = pl.ds(j * _L, _L)
                        v = het[i, sl] - het[_C + i, sl] + re[i, sl] + s * rp[i, sl]
                        nsq = nsq + v * v
                    vec = jnp.where(lane == k, jnp.sum(nsq), vec)
                scores[pl.ds(g * _L, _L)] = _vsqrt(vec)
                return carry2

            lax.fori_loop(0, _C // _L, group_body, 0)
            pltpu.async_copy(scores, out.at[pl.ds(base + loc, _C)], osem)

        issue(sets[0], 0)

        def pair_body(m, carry):
            loc0 = (2 * m) * _C
            drain(sets[0])
            issue(sets[1], loc0 + _C)

            @pl.when(m > 0)
            def _():
                drain_out(sets[0])

            compute(sets[0], loc0)
            drain(sets[1])

            @pl.when(m < nch // 2 - 1)
            def _():
                issue(sets[0], loc0 + 2 * _C)

            @pl.when(m > 0)
            def _():
                drain_out(sets[1])

            compute(sets[1], loc0 + _C)
            return carry

        lax.fori_loop(0, nch // 2, pair_body, 0)
        drain_out(sets[0])
        drain_out(sets[1])

    return trans_d


def kernel(heads, relations, tails, entity_embeddings, entity_projections,
           relation_embeddings, relation_projections):
    h = heads.astype(jnp.int32).reshape(-1, _C)
    t = tails.astype(jnp.int32).reshape(-1, _C)
    idx_ht = jnp.concatenate([h, t], axis=1).reshape(-1)
    k = _build()
    return k(
        idx_ht,
        relations.astype(jnp.int32),
        entity_embeddings,
        entity_projections,
        relation_embeddings,
        relation_projections,
    )


# speedup vs baseline: 1.7339x; 1.0125x over previous
import functools

import jax
import jax.numpy as jnp
from jax import lax
from jax.experimental import pallas as pl
from jax.experimental.pallas import tpu as pltpu
from jax.experimental.pallas import tpu_sc as plsc

_B = 16384
_D = 128
_L = 16
_C = 64
_NC = 2
_NS = 16
_NW = _NC * _NS


def _vsqrt(x):
    xc = jnp.maximum(x, jnp.float32(1e-30))
    bits = plsc.bitcast(xc, jnp.int32)
    seed = plsc.bitcast((bits >> 1) + jnp.int32(0x1FBD1DF5), jnp.float32)
    y = seed
    for _ in range(3):
        y = jnp.float32(0.5) * (y + xc / y)
    return y


@functools.lru_cache(maxsize=1)
def _build():
    bpw = _B // _NW
    nch = bpw // _C
    nsl = _D // _L
    mesh = plsc.VectorSubcoreMesh(
        core_axis_name="c", subcore_axis_name="s",
        num_cores=_NC, num_subcores=_NS)

    def _chunk_scratch():
        return [
            pltpu.VMEM((2 * _C, _D), jnp.float32),
            pltpu.VMEM((2 * _C, _D), jnp.float32),
            pltpu.VMEM((_C, _D), jnp.float32),
            pltpu.VMEM((_C, _D), jnp.float32),
            pltpu.VMEM((_C,), jnp.float32),
            pltpu.SemaphoreType.DMA,
            pltpu.SemaphoreType.DMA,
        ]

    @functools.partial(
        pl.kernel,
        mesh=mesh,
        out_type=jax.ShapeDtypeStruct((_B,), jnp.float32),
        scratch_types=[
            _chunk_scratch(),
            _chunk_scratch(),
            pltpu.VMEM((2 * bpw,), jnp.int32),
            pltpu.VMEM((bpw,), jnp.int32),
            pltpu.SemaphoreType.DMA,
        ],
        compiler_params=pltpu.CompilerParams(needs_layout_passes=False),
    )
    def trans_d(idx_ht, rels, ee, ep, ret, rpt, out,
                set0, set1, idxc, idxr, isem):
        wid = lax.axis_index("s") * _NC + lax.axis_index("c")
        base = wid * bpw
        lane = lax.iota(jnp.int32, _L)
        sets = (set0, set1)

        pltpu.async_copy(idx_ht.at[pl.ds(2 * base, 2 * bpw)], idxc, isem)
        pltpu.async_copy(rels.at[pl.ds(base, bpw)], idxr, isem)
        pltpu.make_async_copy(idx_ht.at[pl.ds(0, 2 * bpw)], idxc, isem).wait()
        pltpu.make_async_copy(rels.at[pl.ds(0, bpw)], idxr, isem).wait()

        def issue(buf, loc):
            het, hpt, re, rp, scores, sem, osem = buf
            iht = idxc.at[pl.ds(2 * loc, 2 * _C)]
            ir = idxr.at[pl.ds(loc, _C)]
            pltpu.async_copy(ee.at[iht], het, sem)
            pltpu.async_copy(ep.at[iht], hpt, sem)
            pltpu.async_copy(ret.at[ir], re, sem)
            pltpu.async_copy(rpt.at[ir], rp, sem)

        def drain(buf):
            het, hpt, re, rp, scores, sem, osem = buf
            for dst in (het, hpt):
                pltpu.make_async_copy(ee.at[pl.ds(0, 2 * _C)], dst, sem).wait()
            for dst in (re, rp):
                pltpu.make_async_copy(ee.at[pl.ds(0, _C)], dst, sem).wait()

        def drain_out(buf):
            het, hpt, re, rp, scores, sem, osem = buf
            pltpu.make_async_copy(out.at[pl.ds(base, _C)], scores, osem).wait()

        def compute(buf, loc):
            het, hpt, re, rp, scores, sem, osem = buf

            def group_body(g, carry2):
                vec = jnp.zeros((_L,), jnp.float32)
                for k in range(_L):
                    i = g * _L + k
                    sl = pl.ds(0, _L)
                    acch = hpt[i, sl] * het[i, sl]
                    acct = hpt[_C + i, sl] * het[_C + i, sl]
                    for j in range(1, nsl):
                        sl = pl.ds(j * _L, _L)
                        acch = acch + hpt[i, sl] * het[i, sl]
                        acct = acct + hpt[_C + i, sl] * het[_C + i, sl]
                    s = jnp.sum(acch - acct)
                    sl = pl.ds(0, _L)
                    v = het[i, sl] - het[_C + i, sl] + re[i, sl] + s * rp[i, sl]
                    nsq = v * v
                    for j in range(1, nsl):
                        sl = pl.ds(j * _L, _L)
                        v = het[i, sl] - het[_C + i, sl] + re[i, sl] + s * rp[i, sl]
                        nsq = nsq + v * v
                    vec = jnp.where(lane == k, jnp.sum(nsq), vec)
                scores[pl.ds(g * _L, _L)] = _vsqrt(vec)
                return carry2

            lax.fori_loop(0, _C // _L, group_body, 0)
            pltpu.async_copy(scores, out.at[pl.ds(base + loc, _C)], osem)

        issue(sets[0], 0)

        def pair_body(m, carry):
            loc0 = (2 * m) * _C
            drain(sets[0])
            issue(sets[1], loc0 + _C)

            @pl.when(m > 0)
            def _():
                drain_out(sets[0])

            compute(sets[0], loc0)
            drain(sets[1])

            @pl.when(m < nch // 2 - 1)
            def _():
                issue(sets[0], loc0 + 2 * _C)

            @pl.when(m > 0)
            def _():
                drain_out(sets[1])

            compute(sets[1], loc0 + _C)
            return carry

        lax.fori_loop(0, nch // 2, pair_body, 0)
        drain_out(sets[0])
        drain_out(sets[1])

    return trans_d


def kernel(heads, relations, tails, entity_embeddings, entity_projections,
           relation_embeddings, relation_projections):
    h = heads.astype(jnp.int32).reshape(-1, _C)
    t = tails.astype(jnp.int32).reshape(-1, _C)
    idx_ht = jnp.concatenate([h, t], axis=1).reshape(-1)
    k = _build()
    return k(
        idx_ht,
        relations.astype(jnp.int32),
        entity_embeddings,
        entity_projections,
        relation_embeddings,
        relation_projections,
    )


# speedup vs baseline: 1.7420x; 1.0047x over previous
import functools

import jax
import jax.numpy as jnp
from jax import lax
from jax.experimental import pallas as pl
from jax.experimental.pallas import tpu as pltpu
from jax.experimental.pallas import tpu_sc as plsc

_B = 16384
_D = 128
_L = 16
_C = 64
_NC = 2
_NS = 16
_NW = _NC * _NS


def _vsqrt(x):
    xc = jnp.maximum(x, jnp.float32(1e-30))
    bits = plsc.bitcast(xc, jnp.int32)
    seed = plsc.bitcast((bits >> 1) + jnp.int32(0x1FBD1DF5), jnp.float32)
    y = seed
    for _ in range(3):
        y = jnp.float32(0.5) * (y + xc / y)
    return y


@functools.lru_cache(maxsize=1)
def _build():
    bpw = _B // _NW
    nch = bpw // _C
    nsl = _D // _L
    mesh = plsc.VectorSubcoreMesh(
        core_axis_name="c", subcore_axis_name="s",
        num_cores=_NC, num_subcores=_NS)

    def _chunk_scratch():
        return [
            pltpu.VMEM((2 * _C, _D), jnp.float32),
            pltpu.VMEM((2 * _C, _D), jnp.float32),
            pltpu.VMEM((_C, _D), jnp.float32),
            pltpu.VMEM((_C, _D), jnp.float32),
            pltpu.VMEM((_C,), jnp.float32),
            pltpu.SemaphoreType.DMA,
            pltpu.SemaphoreType.DMA,
        ]

    @functools.partial(
        pl.kernel,
        mesh=mesh,
        out_type=jax.ShapeDtypeStruct((_B,), jnp.float32),
        scratch_types=[
            _chunk_scratch(),
            _chunk_scratch(),
            pltpu.VMEM((2 * bpw,), jnp.int32),
            pltpu.VMEM((bpw,), jnp.int32),
            pltpu.SemaphoreType.DMA,
        ],
        compiler_params=pltpu.CompilerParams(needs_layout_passes=False),
    )
    def trans_d(idx_ht, rels, ee, ep, ret, rpt, out,
                set0, set1, idxc, idxr, isem):
        wid = lax.axis_index("s") * _NC + lax.axis_index("c")
        base = wid * bpw
        lane = lax.iota(jnp.int32, _L)
        sets = (set0, set1)

        pltpu.async_copy(idx_ht.at[pl.ds(2 * base, 2 * _C)],
                         idxc.at[pl.ds(0, 2 * _C)], isem)
        pltpu.async_copy(rels.at[pl.ds(base, _C)],
                         idxr.at[pl.ds(0, _C)], isem)
        pltpu.make_async_copy(idx_ht.at[pl.ds(0, 2 * _C)],
                              idxc.at[pl.ds(0, 2 * _C)], isem).wait()
        pltpu.make_async_copy(rels.at[pl.ds(0, _C)],
                              idxr.at[pl.ds(0, _C)], isem).wait()

        def issue(buf, loc):
            het, hpt, re, rp, scores, sem, osem = buf
            iht = idxc.at[pl.ds(2 * loc, 2 * _C)]
            ir = idxr.at[pl.ds(loc, _C)]
            pltpu.async_copy(ee.at[iht], het, sem)
            pltpu.async_copy(ep.at[iht], hpt, sem)
            pltpu.async_copy(ret.at[ir], re, sem)
            pltpu.async_copy(rpt.at[ir], rp, sem)

        def drain(buf):
            het, hpt, re, rp, scores, sem, osem = buf
            for dst in (het, hpt):
                pltpu.make_async_copy(ee.at[pl.ds(0, 2 * _C)], dst, sem).wait()
            for dst in (re, rp):
                pltpu.make_async_copy(ee.at[pl.ds(0, _C)], dst, sem).wait()

        def drain_out(buf):
            het, hpt, re, rp, scores, sem, osem = buf
            pltpu.make_async_copy(out.at[pl.ds(base, _C)], scores, osem).wait()

        def compute(buf, loc):
            het, hpt, re, rp, scores, sem, osem = buf

            def group_body(g, carry2):
                vec = jnp.zeros((_L,), jnp.float32)
                for k in range(_L):
                    i = g * _L + k
                    sl = pl.ds(0, _L)
                    acch = hpt[i, sl] * het[i, sl]
                    acct = hpt[_C + i, sl] * het[_C + i, sl]
                    for j in range(1, nsl):
                        sl = pl.ds(j * _L, _L)
                        acch = acch + hpt[i, sl] * het[i, sl]
                        acct = acct + hpt[_C + i, sl] * het[_C + i, sl]
                    s = jnp.sum(acch - acct)
                    sl = pl.ds(0, _L)
                    v = het[i, sl] - het[_C + i, sl] + re[i, sl] + s * rp[i, sl]
                    nsq = v * v
                    for j in range(1, nsl):
                        sl = pl.ds(j * _L, _L)
                        v = het[i, sl] - het[_C + i, sl] + re[i, sl] + s * rp[i, sl]
                        nsq = nsq + v * v
                    vec = jnp.where(lane == k, jnp.sum(nsq), vec)
                scores[pl.ds(g * _L, _L)] = _vsqrt(vec)
                return carry2

            lax.fori_loop(0, _C // _L, group_body, 0)
            pltpu.async_copy(scores, out.at[pl.ds(base + loc, _C)], osem)

        issue(sets[0], 0)

        pltpu.async_copy(idx_ht.at[pl.ds(2 * base + 2 * _C, 2 * (bpw - _C))],
                         idxc.at[pl.ds(2 * _C, 2 * (bpw - _C))], isem)
        pltpu.async_copy(rels.at[pl.ds(base + _C, bpw - _C)],
                         idxr.at[pl.ds(_C, bpw - _C)], isem)
        pltpu.make_async_copy(idx_ht.at[pl.ds(0, 2 * (bpw - _C))],
                              idxc.at[pl.ds(2 * _C, 2 * (bpw - _C))],
                              isem).wait()
        pltpu.make_async_copy(rels.at[pl.ds(0, bpw - _C)],
                              idxr.at[pl.ds(_C, bpw - _C)], isem).wait()
        issue(sets[1], _C)

        def pair_body(m, carry):
            loc0 = (2 * m) * _C
            drain(sets[0])

            @pl.when(m > 0)
            def _():
                drain_out(sets[0])

            compute(sets[0], loc0)

            @pl.when(m < nch // 2 - 1)
            def _():
                issue(sets[0], loc0 + 2 * _C)

            drain(sets[1])

            @pl.when(m > 0)
            def _():
                drain_out(sets[1])

            compute(sets[1], loc0 + _C)

            @pl.when(m < nch // 2 - 1)
            def _():
                issue(sets[1], loc0 + 3 * _C)

            return carry

        lax.fori_loop(0, nch // 2, pair_body, 0)
        drain_out(sets[0])
        drain_out(sets[1])

    return trans_d


def kernel(heads, relations, tails, entity_embeddings, entity_projections,
           relation_embeddings, relation_projections):
    h = heads.astype(jnp.int32).reshape(-1, _C)
    t = tails.astype(jnp.int32).reshape(-1, _C)
    idx_ht = jnp.concatenate([h, t], axis=1).reshape(-1)
    k = _build()
    return k(
        idx_ht,
        relations.astype(jnp.int32),
        entity_embeddings,
        entity_projections,
        relation_embeddings,
        relation_projections,
    )


# speedup vs baseline: 1.7437x; 1.0010x over previous
import functools

import jax
import jax.numpy as jnp
from jax import lax
from jax.experimental import pallas as pl
from jax.experimental.pallas import tpu as pltpu
from jax.experimental.pallas import tpu_sc as plsc

_B = 16384
_D = 128
_L = 16
_C = 64
_NC = 2
_NS = 16
_NW = _NC * _NS


def _vsqrt(x):
    xc = jnp.maximum(x, jnp.float32(1e-30))
    bits = plsc.bitcast(xc, jnp.int32)
    seed = plsc.bitcast((bits >> 1) + jnp.int32(0x1FBD1DF5), jnp.float32)
    y = seed
    for _ in range(3):
        y = jnp.float32(0.5) * (y + xc / y)
    return y


@functools.lru_cache(maxsize=1)
def _build():
    bpw = _B // _NW
    nch = bpw // _C
    nsl = _D // _L
    mesh = plsc.VectorSubcoreMesh(
        core_axis_name="c", subcore_axis_name="s",
        num_cores=_NC, num_subcores=_NS)

    def _chunk_scratch():
        return [
            pltpu.VMEM((2 * _C, _D), jnp.float32),
            pltpu.VMEM((2 * _C, _D), jnp.float32),
            pltpu.VMEM((_C, _D), jnp.float32),
            pltpu.VMEM((_C, _D), jnp.float32),
            pltpu.VMEM((_C,), jnp.float32),
            pltpu.SemaphoreType.DMA,
            pltpu.SemaphoreType.DMA,
        ]

    @functools.partial(
        pl.kernel,
        mesh=mesh,
        out_type=jax.ShapeDtypeStruct((_B,), jnp.float32),
        scratch_types=[
            _chunk_scratch(),
            _chunk_scratch(),
            pltpu.VMEM((2 * bpw,), jnp.int32),
            pltpu.VMEM((bpw,), jnp.int32),
            pltpu.SemaphoreType.DMA,
        ],
        compiler_params=pltpu.CompilerParams(needs_layout_passes=False),
    )
    def trans_d(idx_ht, rels, ee, ep, ret, rpt, out,
                set0, set1, idxc, idxr, isem):
        wid = lax.axis_index("s") * _NC + lax.axis_index("c")
        base = wid * bpw
        lane = lax.iota(jnp.int32, _L)
        sets = (set0, set1)

        pltpu.async_copy(idx_ht.at[pl.ds(2 * base, 2 * _C)],
                         idxc.at[pl.ds(0, 2 * _C)], isem)
        pltpu.async_copy(rels.at[pl.ds(base, _C)],
                         idxr.at[pl.ds(0, _C)], isem)
        pltpu.make_async_copy(idx_ht.at[pl.ds(0, 2 * _C)],
                              idxc.at[pl.ds(0, 2 * _C)], isem).wait()
        pltpu.make_async_copy(rels.at[pl.ds(0, _C)],
                              idxr.at[pl.ds(0, _C)], isem).wait()

        def issue(buf, loc):
            het, hpt, re, rp, scores, sem, osem = buf
            iht = idxc.at[pl.ds(2 * loc, 2 * _C)]
            ir = idxr.at[pl.ds(loc, _C)]
            pltpu.async_copy(ee.at[iht], het, sem)
            pltpu.async_copy(ep.at[iht], hpt, sem)
            pltpu.async_copy(ret.at[ir], re, sem)
            pltpu.async_copy(rpt.at[ir], rp, sem)

        def drain(buf):
            het, hpt, re, rp, scores, sem, osem = buf
            for dst in (het, hpt):
                pltpu.make_async_copy(ee.at[pl.ds(0, 2 * _C)], dst, sem).wait()
            for dst in (re, rp):
                pltpu.make_async_copy(ee.at[pl.ds(0, _C)], dst, sem).wait()

        def drain_out(buf):
            het, hpt, re, rp, scores, sem, osem = buf
            pltpu.make_async_copy(out.at[pl.ds(base, _C)], scores, osem).wait()

        def compute(buf, loc):
            het, hpt, re, rp, scores, sem, osem = buf

            @plsc.parallel_loop(0, _C // _L)
            def group_body(g):
                vec = jnp.zeros((_L,), jnp.float32)
                for k in range(_L):
                    i = g * _L + k
                    sl = pl.ds(0, _L)
                    acch = hpt[i, sl] * het[i, sl]
                    acct = hpt[_C + i, sl] * het[_C + i, sl]
                    for j in range(1, nsl):
                        sl = pl.ds(j * _L, _L)
                        acch = acch + hpt[i, sl] * het[i, sl]
                        acct = acct + hpt[_C + i, sl] * het[_C + i, sl]
                    s = jnp.sum(acch - acct)
                    sl = pl.ds(0, _L)
                    v = het[i, sl] - het[_C + i, sl] + re[i, sl] + s * rp[i, sl]
                    nsq = v * v
                    for j in range(1, nsl):
                        sl = pl.ds(j * _L, _L)
                        v = het[i, sl] - het[_C + i, sl] + re[i, sl] + s * rp[i, sl]
                        nsq = nsq + v * v
                    vec = jnp.where(lane == k, jnp.sum(nsq), vec)
                scores[pl.ds(g * _L, _L)] = _vsqrt(vec)

            pltpu.async_copy(scores, out.at[pl.ds(base + loc, _C)], osem)

        issue(sets[0], 0)

        pltpu.async_copy(idx_ht.at[pl.ds(2 * base + 2 * _C, 2 * (bpw - _C))],
                         idxc.at[pl.ds(2 * _C, 2 * (bpw - _C))], isem)
        pltpu.async_copy(rels.at[pl.ds(base + _C, bpw - _C)],
                         idxr.at[pl.ds(_C, bpw - _C)], isem)
        pltpu.make_async_copy(idx_ht.at[pl.ds(0, 2 * (bpw - _C))],
                              idxc.at[pl.ds(2 * _C, 2 * (bpw - _C))],
                              isem).wait()
        pltpu.make_async_copy(rels.at[pl.ds(0, bpw - _C)],
                              idxr.at[pl.ds(_C, bpw - _C)], isem).wait()
        issue(sets[1], _C)

        def pair_body(m, carry):
            loc0 = (2 * m) * _C
            drain(sets[0])

            @pl.when(m > 0)
            def _():
                drain_out(sets[0])

            compute(sets[0], loc0)

            @pl.when(m < nch // 2 - 1)
            def _():
                issue(sets[0], loc0 + 2 * _C)

            drain(sets[1])

            @pl.when(m > 0)
            def _():
                drain_out(sets[1])

            compute(sets[1], loc0 + _C)

            @pl.when(m < nch // 2 - 1)
            def _():
                issue(sets[1], loc0 + 3 * _C)

            return carry

        lax.fori_loop(0, nch // 2, pair_body, 0)
        drain_out(sets[0])
        drain_out(sets[1])

    return trans_d


def kernel(heads, relations, tails, entity_embeddings, entity_projections,
           relation_embeddings, relation_projections):
    h = heads.astype(jnp.int32).reshape(-1, _C)
    t = tails.astype(jnp.int32).reshape(-1, _C)
    idx_ht = jnp.concatenate([h, t], axis=1).reshape(-1)
    k = _build()
    return k(
        idx_ht,
        relations.astype(jnp.int32),
        entity_embeddings,
        entity_projections,
        relation_embeddings,
        relation_projections,
    )
